# baseline (speedup 1.0000x reference)
import functools

import jax
import jax.numpy as jnp
from jax import lax
from jax.experimental import pallas as pl
from jax.experimental.pallas import tpu as pltpu
from jax.experimental.pallas import tpu_sc as plsc

_N, _E, _D, _H = 10000, 160000, 256, 512
_NC, _NS, _L = 2, 16, 16
_NW = _NC * _NS
_RPB = 160
_NB = 2 * _NW
_RPT = 2 * _RPB
_NPAD = _NB * _RPB
_FLUSH = 512
_EPB = 161792
_BT = 1600
_BLK = 2048
_BG = 32
_TCB = 512
_HW = _H // 2


def _wid():
    return lax.axis_index("s") * _NC + lax.axis_index("c")


def _prepass_fn(src_hbm, dst_hbm, bs_hbm, bdl_hbm, cnt_hbm, deg_hbm,
                ebs, ebd, stg_s0, stg_d0, stg_s1, stg_d1, dega, cntv):
    w = _wid()
    lo = w * _RPT
    zf = jnp.zeros((_L,), jnp.float32)
    for i in range(_RPT // _L):
        dega[pl.ds(i * _L, _L)] = zf
    ones = jnp.ones((_L,), jnp.float32)
    stages = ((stg_s0, stg_d0), (stg_s1, stg_d1))

    def flush(half):
        stg_s, stg_d = stages[half]

        def do(args):
            off, fl = args
            base = (2 * w + half) * _EPB + fl
            pltpu.sync_copy(stg_s.at[pl.ds(0, _FLUSH)],
                            bs_hbm.at[pl.ds(pl.multiple_of(base, 8), _FLUSH)])
            pltpu.sync_copy(stg_d.at[pl.ds(0, _FLUSH)],
                            bdl_hbm.at[pl.ds(pl.multiple_of(base, 8), _FLUSH)])
            rs = stg_s[pl.ds(_FLUSH, _L)]
            rd = stg_d[pl.ds(_FLUSH, _L)]
            stg_s[pl.ds(0, _L)] = rs
            stg_d[pl.ds(0, _L)] = rd
            return off - _FLUSH, fl + _FLUSH

        return do

    def batch(b, carry):
        pltpu.sync_copy(src_hbm.at[pl.ds(b * _BT, _BT)], ebs)
        pltpu.sync_copy(dst_hbm.at[pl.ds(b * _BT, _BT)], ebd)

        def step(j, c2):
            off0, fl0, off1, fl1 = c2
            s16 = ebs[pl.ds(j * _L, _L)]
            d16 = ebd[pl.ds(j * _L, _L)]
            dl = d16 - lo
            m0 = (d16 >= lo) & (d16 < lo + _RPB)
            m1 = (d16 >= lo + _RPB) & (d16 < lo + _RPT)
            plsc.addupdate_scatter(dega, [dl], ones, mask=m0 | m1)
            plsc.store_compressed(stg_s0.at[pl.ds(off0, _L)], s16, mask=m0)
            plsc.store_compressed(stg_d0.at[pl.ds(off0, _L)], dl, mask=m0)
            off0 = off0 + jnp.sum(m0.astype(jnp.int32))
            plsc.store_compressed(stg_s1.at[pl.ds(off1, _L)], s16, mask=m1)
            plsc.store_compressed(stg_d1.at[pl.ds(off1, _L)], dl - _RPB,
                                  mask=m1)
            off1 = off1 + jnp.sum(m1.astype(jnp.int32))
            off0, fl0 = lax.cond(off0 >= _FLUSH, flush(0), lambda a: a,
                                 (off0, fl0))
            off1, fl1 = lax.cond(off1 >= _FLUSH, flush(1), lambda a: a,
                                 (off1, fl1))
            return off0, fl0, off1, fl1

        return lax.fori_loop(0, _BT // _L, step, carry)

    z = jnp.int32(0)
    offs = lax.fori_loop(0, _E // _BT, batch, (z, z, z, z))

    sent_d = jnp.full((_L,), _RPB, dtype=jnp.int32)
    zs = jnp.zeros((_L,), jnp.int32)
    for half in range(2):
        stg_s, stg_d = stages[half]
        off, fl = offs[2 * half], offs[2 * half + 1]

        def fill(k, _):
            stg_s[pl.ds(off + k * _L, _L)] = zs
            stg_d[pl.ds(off + k * _L, _L)] = sent_d
            return 0

        lax.fori_loop(0, (_FLUSH - off + _L - 1) // _L, fill, 0)
        base = (2 * w + half) * _EPB + fl
        pltpu.sync_copy(stg_s.at[pl.ds(0, _FLUSH)],
                        bs_hbm.at[pl.ds(pl.multiple_of(base, 8), _FLUSH)])
        pltpu.sync_copy(stg_d.at[pl.ds(0, _FLUSH)],
                        bdl_hbm.at[pl.ds(pl.multiple_of(base, 8), _FLUSH)])
        cntv[...] = jnp.broadcast_to(fl + _FLUSH, (_L,)).astype(jnp.int32)
        cbase = (2 * w + half) * _L
        pltpu.sync_copy(cntv.at[pl.ds(0, _L)],
                        cnt_hbm.at[pl.ds(pl.multiple_of(cbase, 8), _L)])
    pltpu.sync_copy(dega.at[pl.ds(0, _RPT)],
                    deg_hbm.at[pl.ds(pl.multiple_of(w * _RPT, 8), _RPT)])


def _aggregate_fn(hp, hpb, bs_hbm, bdl_hbm, cnt_hbm, ag,
                  acc, rows_a, rows_b, rows_c, rows_d, sblk, dblk, csm,
                  sem_a, sem_b, sem_c, sem_d):
    w = _wid()
    iota = jnp.arange(_L, dtype=jnp.int32)

    for half in range(2):
        b = 2 * w + half
        lo_b = w * _RPT + half * _RPB
        pltpu.sync_copy(cnt_hbm.at[pl.ds(pl.multiple_of(b * _L, 8), _L)], csm)
        padded = csm[pl.ds(0, _L)][0]
        nblk = lax.shift_right_logical(padded + _BLK - 1, 11)

        def block(b_i, _):
            base = b * _EPB + b_i * _BLK
            pltpu.sync_copy(bs_hbm.at[pl.ds(pl.multiple_of(base, 8), _BLK)],
                            sblk)
            pltpu.sync_copy(bdl_hbm.at[pl.ds(pl.multiple_of(base, 8), _BLK)],
                            dblk)
            nsb = lax.shift_right_logical(
                jnp.minimum(_BLK, padded - b_i * _BLK), 5)
            lax.cond(
                b_i == 0,
                lambda _x: pltpu.sync_copy(hp.at[pl.ds(lo_b, _RPB), :],
                                           acc.at[pl.ds(0, _RPB), :]),
                lambda _x: pltpu.sync_copy(ag.at[pl.ds(lo_b, _RPB), :],
                                           acc.at[pl.ds(0, _RPB), :]),
                0)

            def fire(sb, rbuf, sem):
                pltpu.async_copy(hpb.at[sblk.at[pl.ds(sb * _BG, _BG)]],
                                 rbuf, sem)

            def drain(rbuf, sem):
                pltpu.make_async_copy(hpb.at[pl.ds(0, _BG), :], rbuf, sem
                                      ).wait()

            def compute(sb, rbuf):
                def group(gi, _):
                    dl16 = dblk[pl.ds(sb * _BG + gi * _L, _L)]
                    ev = iota + gi * _L

                    def c16(ci, _):
                        cb = jnp.broadcast_to(ci * 16, (_L,)).astype(jnp.int32)
                        for w0 in range(0, 16, 4):
                            vs = [plsc.load_gather(rbuf, [ev, cb + (w0 + k)])
                                  for k in range(4)]
                            for k in range(4):
                                pb = plsc.bitcast(vs[k], jnp.bfloat16)
                                ae, ao = plsc.unpack(
                                    pb, format=plsc.PackFormat.INTERLEAVED)
                                cv2 = (cb + (w0 + k)) * 2
                                plsc.addupdate_scatter(acc, [dl16, cv2], ae)
                                plsc.addupdate_scatter(acc, [dl16, cv2 + 1],
                                                       ao)
                        return 0

                    lax.fori_loop(0, _HW // 16, c16, 0)
                    return 0

                lax.fori_loop(0, _BG // _L, group, 0)

            bufs = ((rows_a, sem_a), (rows_b, sem_b),
                    (rows_c, sem_c), (rows_d, sem_d))
            for k in range(4):
                fire(k, *bufs[k])

            def quad(q, _):
                sb = 4 * q

                @pl.when(sb < nsb)
                def _():
                    for k in range(4):
                        rbuf, sem = bufs[k]
                        drain(rbuf, sem)
                        compute(sb + k, rbuf)

                        @pl.when(sb + 4 + k < nsb)
                        def _():
                            fire(sb + 4 + k, rbuf, sem)

                return 0

            lax.fori_loop(0, _BLK // _BG // 4, quad, 0)
            pltpu.sync_copy(acc.at[pl.ds(0, _RPB), :],
                            ag.at[pl.ds(lo_b, _RPB), :])
            return 0

        lax.fori_loop(0, nblk, block, 0)


def _tc1_body(xb, w1e, be2, degb, o_t, o_h, o_hb):
    dis = lax.rsqrt(degb[...] + 1.0)
    m = jnp.dot(xb[...], w1e[...], preferred_element_type=jnp.float32)
    hp = m[:, :_H] * dis
    o_h[...] = hp
    o_hb[...] = hp.astype(jnp.bfloat16)
    o_t[...] = m[:, _H:] + be2[...]


def _tc2_body(ab, degb, b1b, w2, o_h, o_hb):
    dis = lax.rsqrt(degb[...] + 1.0)
    h = jnp.maximum(ab[...] * dis + b1b[...], 0.0)
    hp = jnp.dot(h, w2[...], preferred_element_type=jnp.float32) * dis
    o_h[...] = hp
    o_hb[...] = hp.astype(jnp.bfloat16)


def _tc3_body(ab, degb, b2b, tb, wnb, bnb, ob):
    dis = lax.rsqrt(degb[...] + 1.0)
    h = jnp.maximum(ab[...] * dis + b2b[...], 0.0) + tb[...]
    ob[...] = jnp.sum(h * wnb[...], axis=1, keepdims=True) + bnb[0, 0]


@functools.lru_cache(maxsize=None)
def _sc_kernels():
    mesh = plsc.VectorSubcoreMesh(
        core_axis_name="c", subcore_axis_name="s",
        num_cores=_NC, num_subcores=_NS,
    )
    prepass = pl.kernel(
        _prepass_fn,
        out_type=[
            jax.ShapeDtypeStruct((_NB * _EPB,), jnp.int32),
            jax.ShapeDtypeStruct((_NB * _EPB,), jnp.int32),
            jax.ShapeDtypeStruct((_NB * _L,), jnp.int32),
            jax.ShapeDtypeStruct((_NPAD,), jnp.float32),
        ],
        mesh=mesh,
        compiler_params=pltpu.CompilerParams(needs_layout_passes=False),
        scratch_types=[
            pltpu.VMEM((_BT,), jnp.int32),
            pltpu.VMEM((_BT,), jnp.int32),
            pltpu.VMEM((_FLUSH + 2 * _L,), jnp.int32),
            pltpu.VMEM((_FLUSH + 2 * _L,), jnp.int32),
            pltpu.VMEM((_FLUSH + 2 * _L,), jnp.int32),
            pltpu.VMEM((_FLUSH + 2 * _L,), jnp.int32),
            pltpu.VMEM((_RPT,), jnp.float32),
            pltpu.VMEM((_L,), jnp.int32),
        ],
    )
    aggregate = pl.kernel(
        _aggregate_fn,
        out_type=jax.ShapeDtypeStruct((_NPAD, _H), jnp.float32),
        mesh=mesh,
        compiler_params=pltpu.CompilerParams(needs_layout_passes=False,
                                             use_tc_tiling_on_sc=False),
        scratch_types=[
            pltpu.VMEM((_RPB + 1, _H), jnp.float32),
            pltpu.VMEM((_BG, _HW), jnp.int32),
            pltpu.VMEM((_BG, _HW), jnp.int32),
            pltpu.VMEM((_BG, _HW), jnp.int32),
            pltpu.VMEM((_BG, _HW), jnp.int32),
            pltpu.VMEM((_BLK,), jnp.int32),
            pltpu.VMEM((_BLK,), jnp.int32),
            pltpu.VMEM((_L,), jnp.int32),
            pltpu.SemaphoreType.DMA,
            pltpu.SemaphoreType.DMA,
            pltpu.SemaphoreType.DMA,
            pltpu.SemaphoreType.DMA,
        ],
    )
    return prepass, aggregate


def _row_spec(cols):
    return pl.BlockSpec((_TCB, cols), lambda i: (i, 0))


def _const_spec(shape):
    return pl.BlockSpec(shape, lambda i: (0, 0))


def kernel(x, edge_index, W1, b1, W2, b2, We, be, Wn, bn):
    src = edge_index[0]
    dst = edge_index[1]
    xp = jnp.pad(x, ((0, _NPAD - _N), (0, 0)))
    w1e = jnp.concatenate([W1, We], axis=1)

    _prepass, _aggregate = _sc_kernels()
    bs, bdl, cnt, deg_b = _prepass(src, dst)
    deg = deg_b.reshape(_NPAD, 1)

    grid = (_NPAD // _TCB,)
    t, h1p, h1b = pl.pallas_call(
        _tc1_body,
        grid=grid,
        in_specs=[_row_spec(_D), _const_spec((_D, 2 * _H)),
                  _const_spec((1, _H)), _row_spec(1)],
        out_specs=[_row_spec(_H), _row_spec(_H), _row_spec(_H)],
        out_shape=[jax.ShapeDtypeStruct((_NPAD, _H), jnp.float32)] * 2
        + [jax.ShapeDtypeStruct((_NPAD, _H), jnp.bfloat16)],
    )(xp, w1e, be.reshape(1, _H), deg)

    h1w = lax.bitcast_convert_type(h1b.reshape(_NPAD, _HW, 2), jnp.int32)
    agg1 = _aggregate(h1p, h1w, bs, bdl, cnt)

    h2p, h2b = pl.pallas_call(
        _tc2_body,
        grid=grid,
        in_specs=[_row_spec(_H), _row_spec(1), _const_spec((1, _H)),
                  _const_spec((_H, _H))],
        out_specs=[_row_spec(_H), _row_spec(_H)],
        out_shape=[jax.ShapeDtypeStruct((_NPAD, _H), jnp.float32),
                   jax.ShapeDtypeStruct((_NPAD, _H), jnp.bfloat16)],
    )(agg1, deg, b1.reshape(1, _H), W2)

    h2w = lax.bitcast_convert_type(h2b.reshape(_NPAD, _HW, 2), jnp.int32)
    agg2 = _aggregate(h2p, h2w, bs, bdl, cnt)

    out = pl.pallas_call(
        _tc3_body,
        grid=grid,
        in_specs=[_row_spec(_H), _row_spec(1), _const_spec((1, _H)),
                  _row_spec(_H), _const_spec((1, _H)), _const_spec((1, 1))],
        out_specs=_row_spec(1),
        out_shape=jax.ShapeDtypeStruct((_NPAD, 1), jnp.float32),
    )(agg2, deg, b2.reshape(1, _H), t, Wn.reshape(1, _H), bn.reshape(1, 1))

    return out[:_N]


# speedup vs baseline: 1.0177x; 1.0177x over previous
import functools

import jax
import jax.numpy as jnp
from jax import lax
from jax.experimental import pallas as pl
from jax.experimental.pallas import tpu as pltpu
from jax.experimental.pallas import tpu_sc as plsc

_N, _E, _D, _H = 10000, 160000, 256, 512
_NC, _NS, _L = 2, 16, 16
_NW = _NC * _NS
_RPB = 160
_NB = 2 * _NW
_RPT = 2 * _RPB
_NPAD = _NB * _RPB
_FLUSH = 512
_EPB = 161792
_BT = 1600
_BLK = 2048
_BG = 32
_TCB = 512
_HW = _H // 2


def _wid():
    return lax.axis_index("s") * _NC + lax.axis_index("c")


def _prepass_fn(src_hbm, dst_hbm, bs_hbm, bdl_hbm, cnt_hbm, deg_hbm,
                ebs, ebd, stg_s0, stg_d0, stg_s1, stg_d1, dega, cntv):
    w = _wid()
    lo = w * _RPT
    zf = jnp.zeros((_L,), jnp.float32)
    for i in range(_RPT // _L):
        dega[pl.ds(i * _L, _L)] = zf
    ones = jnp.ones((_L,), jnp.float32)
    stages = ((stg_s0, stg_d0), (stg_s1, stg_d1))

    def flush(half):
        stg_s, stg_d = stages[half]

        def do(args):
            off, fl = args
            base = (2 * w + half) * _EPB + fl
            pltpu.sync_copy(stg_s.at[pl.ds(0, _FLUSH)],
                            bs_hbm.at[pl.ds(pl.multiple_of(base, 8), _FLUSH)])
            pltpu.sync_copy(stg_d.at[pl.ds(0, _FLUSH)],
                            bdl_hbm.at[pl.ds(pl.multiple_of(base, 8), _FLUSH)])
            rs = stg_s[pl.ds(_FLUSH, _L)]
            rd = stg_d[pl.ds(_FLUSH, _L)]
            stg_s[pl.ds(0, _L)] = rs
            stg_d[pl.ds(0, _L)] = rd
            return off - _FLUSH, fl + _FLUSH

        return do

    def batch(b, carry):
        pltpu.sync_copy(src_hbm.at[pl.ds(b * _BT, _BT)], ebs)
        pltpu.sync_copy(dst_hbm.at[pl.ds(b * _BT, _BT)], ebd)

        def step(j, c2):
            off0, fl0, off1, fl1 = c2
            s16 = ebs[pl.ds(j * _L, _L)]
            d16 = ebd[pl.ds(j * _L, _L)]
            dl = d16 - lo
            m0 = (d16 >= lo) & (d16 < lo + _RPB)
            m1 = (d16 >= lo + _RPB) & (d16 < lo + _RPT)
            plsc.addupdate_scatter(dega, [dl], ones, mask=m0 | m1)
            plsc.store_compressed(stg_s0.at[pl.ds(off0, _L)], s16, mask=m0)
            plsc.store_compressed(stg_d0.at[pl.ds(off0, _L)], dl, mask=m0)
            off0 = off0 + plsc.all_reduce_population_count(m0)[0]
            plsc.store_compressed(stg_s1.at[pl.ds(off1, _L)], s16, mask=m1)
            plsc.store_compressed(stg_d1.at[pl.ds(off1, _L)], dl - _RPB,
                                  mask=m1)
            off1 = off1 + plsc.all_reduce_population_count(m1)[0]
            off0, fl0 = lax.cond(off0 >= _FLUSH, flush(0), lambda a: a,
                                 (off0, fl0))
            off1, fl1 = lax.cond(off1 >= _FLUSH, flush(1), lambda a: a,
                                 (off1, fl1))
            return off0, fl0, off1, fl1

        return lax.fori_loop(0, _BT // _L, step, carry)

    z = jnp.int32(0)
    offs = lax.fori_loop(0, _E // _BT, batch, (z, z, z, z))

    sent_d = jnp.full((_L,), _RPB, dtype=jnp.int32)
    zs = jnp.zeros((_L,), jnp.int32)
    for half in range(2):
        stg_s, stg_d = stages[half]
        off, fl = offs[2 * half], offs[2 * half + 1]

        def fill(k, _):
            stg_s[pl.ds(off + k * _L, _L)] = zs
            stg_d[pl.ds(off + k * _L, _L)] = sent_d
            return 0

        lax.fori_loop(0, (_FLUSH - off + _L - 1) // _L, fill, 0)
        base = (2 * w + half) * _EPB + fl
        pltpu.sync_copy(stg_s.at[pl.ds(0, _FLUSH)],
                        bs_hbm.at[pl.ds(pl.multiple_of(base, 8), _FLUSH)])
        pltpu.sync_copy(stg_d.at[pl.ds(0, _FLUSH)],
                        bdl_hbm.at[pl.ds(pl.multiple_of(base, 8), _FLUSH)])
        cntv[...] = jnp.broadcast_to(fl + _FLUSH, (_L,)).astype(jnp.int32)
        cbase = (2 * w + half) * _L
        pltpu.sync_copy(cntv.at[pl.ds(0, _L)],
                        cnt_hbm.at[pl.ds(pl.multiple_of(cbase, 8), _L)])
    pltpu.sync_copy(dega.at[pl.ds(0, _RPT)],
                    deg_hbm.at[pl.ds(pl.multiple_of(w * _RPT, 8), _RPT)])


def _aggregate_fn(hp, hpb, bs_hbm, bdl_hbm, cnt_hbm, ag,
                  acc, rows_a, rows_b, rows_c, rows_d, sblk, dblk, csm,
                  sem_a, sem_b, sem_c, sem_d):
    w = _wid()
    iota = jnp.arange(_L, dtype=jnp.int32)

    for half in range(2):
        b = 2 * w + half
        lo_b = w * _RPT + half * _RPB
        pltpu.sync_copy(cnt_hbm.at[pl.ds(pl.multiple_of(b * _L, 8), _L)], csm)
        padded = csm[pl.ds(0, _L)][0]
        nblk = lax.shift_right_logical(padded + _BLK - 1, 11)

        def block(b_i, _):
            base = b * _EPB + b_i * _BLK
            pltpu.sync_copy(bs_hbm.at[pl.ds(pl.multiple_of(base, 8), _BLK)],
                            sblk)
            pltpu.sync_copy(bdl_hbm.at[pl.ds(pl.multiple_of(base, 8), _BLK)],
                            dblk)
            nsb = lax.shift_right_logical(
                jnp.minimum(_BLK, padded - b_i * _BLK), 5)
            lax.cond(
                b_i == 0,
                lambda _x: pltpu.sync_copy(hp.at[pl.ds(lo_b, _RPB), :],
                                           acc.at[pl.ds(0, _RPB), :]),
                lambda _x: pltpu.sync_copy(ag.at[pl.ds(lo_b, _RPB), :],
                                           acc.at[pl.ds(0, _RPB), :]),
                0)

            def fire(sb, rbuf, sem):
                pltpu.async_copy(hpb.at[sblk.at[pl.ds(sb * _BG, _BG)]],
                                 rbuf, sem)

            def drain(rbuf, sem):
                pltpu.make_async_copy(hpb.at[pl.ds(0, _BG), :], rbuf, sem
                                      ).wait()

            def compute(sb, rbuf):
                def group(gi, _):
                    dl16 = dblk[pl.ds(sb * _BG + gi * _L, _L)]
                    ev = iota + gi * _L

                    def c16(ci, _):
                        cb = jnp.broadcast_to(ci * 16, (_L,)).astype(jnp.int32)
                        for w0 in range(0, 16, 4):
                            vs = [plsc.load_gather(rbuf, [ev, cb + (w0 + k)])
                                  for k in range(4)]
                            for k in range(4):
                                pb = plsc.bitcast(vs[k], jnp.bfloat16)
                                ae, ao = plsc.unpack(
                                    pb, format=plsc.PackFormat.INTERLEAVED)
                                cv2 = (cb + (w0 + k)) * 2
                                plsc.addupdate_scatter(acc, [dl16, cv2], ae)
                                plsc.addupdate_scatter(acc, [dl16, cv2 + 1],
                                                       ao)
                        return 0

                    lax.fori_loop(0, _HW // 16, c16, 0)
                    return 0

                lax.fori_loop(0, _BG // _L, group, 0)

            bufs = ((rows_a, sem_a), (rows_b, sem_b),
                    (rows_c, sem_c), (rows_d, sem_d))
            for k in range(4):
                fire(k, *bufs[k])

            def quad(q, _):
                sb = 4 * q

                @pl.when(sb < nsb)
                def _():
                    for k in range(4):
                        rbuf, sem = bufs[k]
                        drain(rbuf, sem)
                        compute(sb + k, rbuf)

                        @pl.when(sb + 4 + k < nsb)
                        def _():
                            fire(sb + 4 + k, rbuf, sem)

                return 0

            lax.fori_loop(0, _BLK // _BG // 4, quad, 0)
            pltpu.sync_copy(acc.at[pl.ds(0, _RPB), :],
                            ag.at[pl.ds(lo_b, _RPB), :])
            return 0

        lax.fori_loop(0, nblk, block, 0)


def _tc1_body(xb, w1e, be2, degb, o_t, o_h, o_hb):
    dis = lax.rsqrt(degb[...] + 1.0)
    m = jnp.dot(xb[...], w1e[...], preferred_element_type=jnp.float32)
    hp = m[:, :_H] * dis
    o_h[...] = hp
    o_hb[...] = hp.astype(jnp.bfloat16)
    o_t[...] = m[:, _H:] + be2[...]


def _tc2_body(ab, degb, b1b, w2, o_h, o_hb):
    dis = lax.rsqrt(degb[...] + 1.0)
    h = jnp.maximum(ab[...] * dis + b1b[...], 0.0)
    hp = jnp.dot(h, w2[...], preferred_element_type=jnp.float32) * dis
    o_h[...] = hp
    o_hb[...] = hp.astype(jnp.bfloat16)


def _tc3_body(ab, degb, b2b, tb, wnb, bnb, ob):
    dis = lax.rsqrt(degb[...] + 1.0)
    h = jnp.maximum(ab[...] * dis + b2b[...], 0.0) + tb[...]
    ob[...] = jnp.sum(h * wnb[...], axis=1, keepdims=True) + bnb[0, 0]


@functools.lru_cache(maxsize=None)
def _sc_kernels():
    mesh = plsc.VectorSubcoreMesh(
        core_axis_name="c", subcore_axis_name="s",
        num_cores=_NC, num_subcores=_NS,
    )
    prepass = pl.kernel(
        _prepass_fn,
        out_type=[
            jax.ShapeDtypeStruct((_NB * _EPB,), jnp.int32),
            jax.ShapeDtypeStruct((_NB * _EPB,), jnp.int32),
            jax.ShapeDtypeStruct((_NB * _L,), jnp.int32),
            jax.ShapeDtypeStruct((_NPAD,), jnp.float32),
        ],
        mesh=mesh,
        compiler_params=pltpu.CompilerParams(needs_layout_passes=False),
        scratch_types=[
            pltpu.VMEM((_BT,), jnp.int32),
            pltpu.VMEM((_BT,), jnp.int32),
            pltpu.VMEM((_FLUSH + 2 * _L,), jnp.int32),
            pltpu.VMEM((_FLUSH + 2 * _L,), jnp.int32),
            pltpu.VMEM((_FLUSH + 2 * _L,), jnp.int32),
            pltpu.VMEM((_FLUSH + 2 * _L,), jnp.int32),
            pltpu.VMEM((_RPT,), jnp.float32),
            pltpu.VMEM((_L,), jnp.int32),
        ],
    )
    aggregate = pl.kernel(
        _aggregate_fn,
        out_type=jax.ShapeDtypeStruct((_NPAD, _H), jnp.float32),
        mesh=mesh,
        compiler_params=pltpu.CompilerParams(needs_layout_passes=False),
        scratch_types=[
            pltpu.VMEM((_RPB + 1, _H), jnp.float32),
            pltpu.VMEM((_BG, _HW), jnp.int32),
            pltpu.VMEM((_BG, _HW), jnp.int32),
            pltpu.VMEM((_BG, _HW), jnp.int32),
            pltpu.VMEM((_BG, _HW), jnp.int32),
            pltpu.VMEM((_BLK,), jnp.int32),
            pltpu.VMEM((_BLK,), jnp.int32),
            pltpu.VMEM((_L,), jnp.int32),
            pltpu.SemaphoreType.DMA,
            pltpu.SemaphoreType.DMA,
            pltpu.SemaphoreType.DMA,
            pltpu.SemaphoreType.DMA,
        ],
    )
    return prepass, aggregate


def _row_spec(cols):
    return pl.BlockSpec((_TCB, cols), lambda i: (i, 0))


def _const_spec(shape):
    return pl.BlockSpec(shape, lambda i: (0, 0))


def kernel(x, edge_index, W1, b1, W2, b2, We, be, Wn, bn):
    src = edge_index[0]
    dst = edge_index[1]
    xp = jnp.pad(x, ((0, _NPAD - _N), (0, 0)))
    w1e = jnp.concatenate([W1, We], axis=1)

    _prepass, _aggregate = _sc_kernels()
    bs, bdl, cnt, deg_b = _prepass(src, dst)
    deg = deg_b.reshape(_NPAD, 1)

    grid = (_NPAD // _TCB,)
    t, h1p, h1b = pl.pallas_call(
        _tc1_body,
        grid=grid,
        in_specs=[_row_spec(_D), _const_spec((_D, 2 * _H)),
                  _const_spec((1, _H)), _row_spec(1)],
        out_specs=[_row_spec(_H), _row_spec(_H), _row_spec(_H)],
        out_shape=[jax.ShapeDtypeStruct((_NPAD, _H), jnp.float32)] * 2
        + [jax.ShapeDtypeStruct((_NPAD, _H), jnp.bfloat16)],
    )(xp, w1e, be.reshape(1, _H), deg)

    h1w = lax.bitcast_convert_type(h1b.reshape(_NPAD, _HW, 2), jnp.int32)
    agg1 = _aggregate(h1p, h1w, bs, bdl, cnt)

    h2p, h2b = pl.pallas_call(
        _tc2_body,
        grid=grid,
        in_specs=[_row_spec(_H), _row_spec(1), _const_spec((1, _H)),
                  _const_spec((_H, _H))],
        out_specs=[_row_spec(_H), _row_spec(_H)],
        out_shape=[jax.ShapeDtypeStruct((_NPAD, _H), jnp.float32),
                   jax.ShapeDtypeStruct((_NPAD, _H), jnp.bfloat16)],
    )(agg1, deg, b1.reshape(1, _H), W2)

    h2w = lax.bitcast_convert_type(h2b.reshape(_NPAD, _HW, 2), jnp.int32)
    agg2 = _aggregate(h2p, h2w, bs, bdl, cnt)

    out = pl.pallas_call(
        _tc3_body,
        grid=grid,
        in_specs=[_row_spec(_H), _row_spec(1), _const_spec((1, _H)),
                  _row_spec(_H), _const_spec((1, _H)), _const_spec((1, 1))],
        out_specs=_row_spec(1),
        out_shape=jax.ShapeDtypeStruct((_NPAD, 1), jnp.float32),
    )(agg2, deg, b2.reshape(1, _H), t, Wn.reshape(1, _H), bn.reshape(1, 1))

    return out[:_N]
